# baseline (speedup 1.0000x reference)
import jax
import jax.numpy as jnp
from jax.experimental import pallas as pl
from jax.experimental.pallas import tpu as pltpu


def _round_up(n, m):
    return ((n + m - 1) // m) * m


def _fused_kernel(x_ref, w1_ref, w2_ref, b1_ref, b2_ref, out_ref):
    w_row_t = jax.lax.dot_general(
        w2_ref[...],
        w1_ref[...],
        dimension_numbers=(((0,), (1,)), ((), ())),
        preferred_element_type=jnp.float32,
    )
    bias = jax.lax.dot_general(
        b1_ref[...],
        w2_ref[...],
        dimension_numbers=(((1,), (0,)), ((), ())),
        preferred_element_type=jnp.float32,
    ) + b2_ref[...]
    y = jax.lax.dot_general(
        w_row_t,
        x_ref[...],
        dimension_numbers=(((1,), (1,)), ((), ())),
        preferred_element_type=jnp.float32,
    )
    out_ref[...] = y + bias


def _coxnet(x, w1, b1, w2, b2, *, tile_b=2048):
    B, D1 = x.shape
    H = w1.shape[1]

    tile_b = min(tile_b, _round_up(max(B, 1), 128))
    b_pad = _round_up(B, tile_b)
    if b_pad != B:
        x = jnp.pad(x, ((0, b_pad - B), (0, 0)))
    grid = (b_pad // tile_b,)

    out_t = pl.pallas_call(
        _fused_kernel,
        out_shape=jax.ShapeDtypeStruct((1, b_pad), jnp.float32),
        grid=grid,
        in_specs=[
            pl.BlockSpec((tile_b, D1), lambda i: (i, 0)),
            pl.BlockSpec((D1, H), lambda i: (0, 0)),
            pl.BlockSpec((H, 1), lambda i: (0, 0)),
            pl.BlockSpec((1, H), lambda i: (0, 0)),
            pl.BlockSpec((1, 1), lambda i: (0, 0)),
        ],
        out_specs=pl.BlockSpec((1, tile_b), lambda i: (0, i)),
        compiler_params=pltpu.CompilerParams(
            dimension_semantics=("parallel",),
        ),
    )(x, w1, w2, b1, b2)

    return out_t[0, :B].reshape(B, 1)


def kernel(x, w1, b1, w2, b2):
    return _coxnet(x, w1, b1, w2, b2)


# speedup vs baseline: 1.2291x; 1.2291x over previous
import jax
import jax.numpy as jnp
from jax.experimental import pallas as pl
from jax.experimental.pallas import tpu as pltpu


def _round_up(n, m):
    return ((n + m - 1) // m) * m


def _fused_kernel(x_ref, w1_ref, w2_ref, b1_ref, b2_ref, out_ref):
    w_row_t = jax.lax.dot_general(
        w2_ref[...],
        w1_ref[...],
        dimension_numbers=(((0,), (1,)), ((), ())),
        preferred_element_type=jnp.float32,
    )
    bias = jax.lax.dot_general(
        b1_ref[...],
        w2_ref[...],
        dimension_numbers=(((1,), (0,)), ((), ())),
        preferred_element_type=jnp.float32,
    ) + b2_ref[...]
    y = jax.lax.dot_general(
        w_row_t,
        x_ref[...],
        dimension_numbers=(((1,), (1,)), ((), ())),
        preferred_element_type=jnp.float32,
    )
    out_ref[...] = y + bias


def _coxnet(x, w1, b1, w2, b2, *, tile_b=8192):
    B, D1 = x.shape
    H = w1.shape[1]

    tile_b = min(tile_b, _round_up(max(B, 1), 128))
    b_pad = _round_up(B, tile_b)
    if b_pad != B:
        x = jnp.pad(x, ((0, b_pad - B), (0, 0)))
    grid = (b_pad // tile_b,)

    out_t = pl.pallas_call(
        _fused_kernel,
        out_shape=jax.ShapeDtypeStruct((1, b_pad), jnp.float32),
        grid=grid,
        in_specs=[
            pl.BlockSpec((tile_b, D1), lambda i: (i, 0)),
            pl.BlockSpec((D1, H), lambda i: (0, 0)),
            pl.BlockSpec((H, 1), lambda i: (0, 0)),
            pl.BlockSpec((1, H), lambda i: (0, 0)),
            pl.BlockSpec((1, 1), lambda i: (0, 0)),
        ],
        out_specs=pl.BlockSpec((1, tile_b), lambda i: (0, i)),
        compiler_params=pltpu.CompilerParams(
            dimension_semantics=("parallel",),
        ),
    )(x, w1, w2, b1, b2)

    return out_t[0, :B].reshape(B, 1)


def kernel(x, w1, b1, w2, b2):
    return _coxnet(x, w1, b1, w2, b2)
